# Initial kernel scaffold; baseline (speedup 1.0000x reference)
import jax
import jax.numpy as jnp
from jax.experimental import pallas as pl


def kernel(source_ufea, target_ufea, source_UV_adj, source_VU_adj, target_UV_adj, target_VU_adj, source_rate, L0_gc1_W, L0_gc1_b, L0_gc2_W, L0_gc2_b, L0_gc3_W, L0_gc3_b, L0_gc4_W, L0_gc4_b, L0_su_W, L0_su_b, L0_tu_W, L0_tu_b, L1_gc1_W, L1_gc1_b, L1_gc2_W, L1_gc2_b, L1_gc3m_W, L1_gc3m_b, L1_gc3s_W, L1_gc3s_b, L1_gc4m_W, L1_gc4m_b, L1_gc4s_W, L1_gc4s_b, L1_sum_W, L1_sum_b, L1_sus_W, L1_sus_b, L1_tum_W, L1_tum_b, L1_tus_W, L1_tus_b):
    raise NotImplementedError("write your pallas kernel here")


# speedup vs baseline: 1.3127x; 1.3127x over previous
import jax
import jax.numpy as jnp
from jax.experimental import pallas as pl
from jax.experimental.pallas import tpu as pltpu

N = 4096
D = 128
ALPHA = 0.1
BM = 512


def _lrelu(x):
    return jnp.where(x > 0, x, ALPHA * x)


def _hop_body(xs_ref, xt_ref, adjs_ref, adjt_ref, w1_ref, w2_ref,
              b1_ref, b2_ref, outs_ref, outt_ref, ys_ref, yt_ref):
    @pl.when(pl.program_id(0) == 0)
    def _():
        ys_ref[...] = jnp.dot(xs_ref[...], w1_ref[...],
                              preferred_element_type=jnp.float32)
        yt_ref[...] = jnp.dot(xt_ref[...], w2_ref[...],
                              preferred_element_type=jnp.float32)

    outs_ref[...] = _lrelu(
        jnp.dot(adjs_ref[...], ys_ref[...],
                preferred_element_type=jnp.float32) + b1_ref[...])
    outt_ref[...] = _lrelu(
        jnp.dot(adjt_ref[...], yt_ref[...],
                preferred_element_type=jnp.float32) + b2_ref[...])


def _mix_body(sho_ref, tho_ref, s_ref, t_ref, adjs_ref, adjt_ref,
              w3_ref, w4_ref, b3_ref, b4_ref,
              wsut_ref, wsub_ref, bsu_ref, wtut_ref, wtub_ref, btu_ref,
              mix_ref, ys_ref, yt_ref):
    @pl.when(pl.program_id(0) == 0)
    def _():
        ys_ref[...] = jnp.dot(sho_ref[...], w3_ref[...],
                              preferred_element_type=jnp.float32)
        yt_ref[...] = jnp.dot(tho_ref[...], w4_ref[...],
                              preferred_element_type=jnp.float32)

    s_ho2 = _lrelu(jnp.dot(adjs_ref[...], ys_ref[...],
                           preferred_element_type=jnp.float32) + b3_ref[...])
    t_ho2 = _lrelu(jnp.dot(adjt_ref[...], yt_ref[...],
                           preferred_element_type=jnp.float32) + b4_ref[...])
    sU = (jnp.dot(s_ho2, wsut_ref[...], preferred_element_type=jnp.float32)
          + jnp.dot(s_ref[...], wsub_ref[...],
                    preferred_element_type=jnp.float32) + bsu_ref[...])
    tU = (jnp.dot(t_ho2, wtut_ref[...], preferred_element_type=jnp.float32)
          + jnp.dot(t_ref[...], wtub_ref[...],
                    preferred_element_type=jnp.float32) + btu_ref[...])
    mix_ref[...] = jnp.maximum(sU, 0.0) + jnp.maximum(tU, 0.0)


def _last_body(b1in_ref, b2in_ref, mix_ref, adjs_ref, adjt_ref,
               w3ms_ref, b3ms_ref, w4ms_ref, b4ms_ref,
               wsm_ref, wtm_ref, wmixm_ref, bm_ref,
               wsl_ref, wtl_ref, wmixl_ref, bl_ref,
               mean_ref, logstd_ref, ys_ref, yt_ref):
    @pl.when(pl.program_id(0) == 0)
    def _():
        ys_ref[...] = jnp.dot(b1in_ref[...], w3ms_ref[...],
                              preferred_element_type=jnp.float32)
        yt_ref[...] = jnp.dot(b2in_ref[...], w4ms_ref[...],
                              preferred_element_type=jnp.float32)

    sml = _lrelu(jnp.dot(adjs_ref[...], ys_ref[...],
                         preferred_element_type=jnp.float32) + b3ms_ref[...])
    tml = _lrelu(jnp.dot(adjt_ref[...], yt_ref[...],
                         preferred_element_type=jnp.float32) + b4ms_ref[...])
    s_m, s_l = sml[:, :D], sml[:, D:]
    t_m, t_l = tml[:, :D], tml[:, D:]
    mixv = mix_ref[...]
    mean_ref[...] = (
        jnp.dot(s_m, wsm_ref[...], preferred_element_type=jnp.float32)
        + jnp.dot(t_m, wtm_ref[...], preferred_element_type=jnp.float32)
        + jnp.dot(mixv, wmixm_ref[...], preferred_element_type=jnp.float32)
        + bm_ref[...])
    logstd_ref[...] = (
        jnp.dot(s_l, wsl_ref[...], preferred_element_type=jnp.float32)
        + jnp.dot(t_l, wtl_ref[...], preferred_element_type=jnp.float32)
        + jnp.dot(mixv, wmixl_ref[...], preferred_element_type=jnp.float32)
        + bl_ref[...])


def _const_spec(shape):
    return pl.BlockSpec(shape, lambda i: (0, 0))


def _row_spec(width):
    return pl.BlockSpec((BM, width), lambda i: (i, 0))


_GRID = N // BM
_PARAMS = pltpu.CompilerParams(dimension_semantics=("arbitrary",))


def _hop_call(xs, xt, adjs, adjt, w1, w2, b1, b2):
    return pl.pallas_call(
        _hop_body,
        grid=(_GRID,),
        in_specs=[
            _const_spec((N, D)), _const_spec((N, D)),
            _row_spec(N), _row_spec(N),
            _const_spec((D, D)), _const_spec((D, D)),
            _const_spec((1, D)), _const_spec((1, D)),
        ],
        out_specs=[_row_spec(D), _row_spec(D)],
        out_shape=[jax.ShapeDtypeStruct((N, D), jnp.float32)] * 2,
        scratch_shapes=[pltpu.VMEM((N, D), jnp.float32)] * 2,
        compiler_params=_PARAMS,
    )(xs, xt, adjs, adjt, w1, w2, b1, b2)


def _mix_call(sho, tho, s, t, adjs, adjt, w3, w4, b3, b4,
              wsut, wsub, bsu, wtut, wtub, btu):
    return pl.pallas_call(
        _mix_body,
        grid=(_GRID,),
        in_specs=[
            _const_spec((N, D)), _const_spec((N, D)),
            _row_spec(D), _row_spec(D),
            _row_spec(N), _row_spec(N),
            _const_spec((D, D)), _const_spec((D, D)),
            _const_spec((1, D)), _const_spec((1, D)),
            _const_spec((D, D)), _const_spec((D, D)), _const_spec((1, D)),
            _const_spec((D, D)), _const_spec((D, D)), _const_spec((1, D)),
        ],
        out_specs=[_row_spec(D)],
        out_shape=[jax.ShapeDtypeStruct((N, D), jnp.float32)],
        scratch_shapes=[pltpu.VMEM((N, D), jnp.float32)] * 2,
        compiler_params=_PARAMS,
    )(sho, tho, s, t, adjs, adjt, w3, w4, b3, b4,
      wsut, wsub, bsu, wtut, wtub, btu)[0]


def _last_call(b1in, b2in, mix, adjs, adjt, w3ms, b3ms, w4ms, b4ms,
               wsm, wtm, wmixm, bm, wsl, wtl, wmixl, bl):
    return pl.pallas_call(
        _last_body,
        grid=(_GRID,),
        in_specs=[
            _const_spec((N, D)), _const_spec((N, D)),
            _row_spec(D),
            _row_spec(N), _row_spec(N),
            _const_spec((D, 2 * D)), _const_spec((1, 2 * D)),
            _const_spec((D, 2 * D)), _const_spec((1, 2 * D)),
            _const_spec((D, D)), _const_spec((D, D)), _const_spec((D, D)),
            _const_spec((1, D)),
            _const_spec((D, D)), _const_spec((D, D)), _const_spec((D, D)),
            _const_spec((1, D)),
        ],
        out_specs=[_row_spec(D), _row_spec(D)],
        out_shape=[jax.ShapeDtypeStruct((N, D), jnp.float32)] * 2,
        scratch_shapes=[pltpu.VMEM((N, 2 * D), jnp.float32)] * 2,
        compiler_params=_PARAMS,
    )(b1in, b2in, mix, adjs, adjt, w3ms, b3ms, w4ms, b4ms,
      wsm, wtm, wmixm, bm, wsl, wtl, wmixl, bl)


def kernel(source_ufea, target_ufea, source_UV_adj, source_VU_adj,
           target_UV_adj, target_VU_adj, source_rate,
           L0_gc1_W, L0_gc1_b, L0_gc2_W, L0_gc2_b, L0_gc3_W, L0_gc3_b,
           L0_gc4_W, L0_gc4_b, L0_su_W, L0_su_b, L0_tu_W, L0_tu_b,
           L1_gc1_W, L1_gc1_b, L1_gc2_W, L1_gc2_b, L1_gc3m_W, L1_gc3m_b,
           L1_gc3s_W, L1_gc3s_b, L1_gc4m_W, L1_gc4m_b, L1_gc4s_W,
           L1_gc4s_b, L1_sum_W, L1_sum_b, L1_sus_W, L1_sus_b, L1_tum_W,
           L1_tum_b, L1_tus_W, L1_tus_b):
    r = source_rate[0]
    rc = 1.0 - r
    row = lambda b: b.reshape(1, -1)

    s_ho, t_ho = _hop_call(
        source_ufea, target_ufea, source_VU_adj, target_VU_adj,
        L0_gc1_W, L0_gc2_W, row(L0_gc1_b), row(L0_gc2_b))

    mix = _mix_call(
        s_ho, t_ho, source_ufea, target_ufea, source_UV_adj, target_UV_adj,
        L0_gc3_W, L0_gc4_W, row(L0_gc3_b), row(L0_gc4_b),
        r * L0_su_W[:D], r * L0_su_W[D:], row(r * L0_su_b),
        rc * L0_tu_W[:D], rc * L0_tu_W[D:], row(rc * L0_tu_b))

    b1s, b2t = _hop_call(
        mix, mix, source_VU_adj, target_VU_adj,
        L1_gc1_W, L1_gc2_W, row(L1_gc1_b), row(L1_gc2_b))

    w3ms = jnp.concatenate([L1_gc3m_W, L1_gc3s_W], axis=1)
    b3ms = jnp.concatenate([L1_gc3m_b, L1_gc3s_b])
    w4ms = jnp.concatenate([L1_gc4m_W, L1_gc4s_W], axis=1)
    b4ms = jnp.concatenate([L1_gc4m_b, L1_gc4s_b])
    mean, logstd = _last_call(
        b1s, b2t, mix, source_UV_adj, target_UV_adj,
        w3ms, row(b3ms), w4ms, row(b4ms),
        r * L1_sum_W[:D], rc * L1_tum_W[:D],
        r * L1_sum_W[D:] + rc * L1_tum_W[D:],
        row(r * L1_sum_b + rc * L1_tum_b),
        r * L1_sus_W[:D], rc * L1_tus_W[:D],
        r * L1_sus_W[D:] + rc * L1_tus_W[D:],
        row(r * L1_sus_b + rc * L1_tus_b))
    return (mean, logstd)


# speedup vs baseline: 1.4591x; 1.1115x over previous
import jax
import jax.numpy as jnp
from jax.experimental import pallas as pl
from jax.experimental.pallas import tpu as pltpu

N = 4096
D = 128
ALPHA = 0.1
BM = 256
GB = N // BM
F32 = jnp.float32


def _lrelu(x):
    return jnp.where(x > 0, x, ALPHA * x)


def _dot(a, b):
    return jnp.dot(a, b, preferred_element_type=F32,
                   precision=jax.lax.Precision.DEFAULT)


def _body(s_ref, t_ref, svu_ref, tvu_ref, suv_ref, tuv_ref,
          w1_ref, b1_ref, w2_ref, b2_ref,
          w3_ref, b3_ref, w4_ref, b4_ref,
          wsut_ref, wsub_ref, bsu_ref, wtut_ref, wtub_ref, btu_ref,
          w5_ref, b5_ref, w6_ref, b6_ref,
          w3ms_ref, b3ms_ref, w4ms_ref, b4ms_ref,
          wsm_ref, wtm_ref, wmixm_ref, bm_ref,
          wsl_ref, wtl_ref, wmixl_ref, bl_ref,
          mean_ref, logstd_ref,
          sho_ref, tho_ref, mix_ref, bb1_ref, bb2_ref, ys_ref, yt_ref):
    g = pl.program_id(0)
    stage = g // GB
    i = g % GB
    rows = pl.ds(i * BM, BM)

    @pl.when(stage == 0)
    def _():
        @pl.when(i == 0)
        def _():
            ys_ref[:, :D] = _dot(s_ref[...], w1_ref[...])
            yt_ref[:, :D] = _dot(t_ref[...], w2_ref[...])

        sho_ref[rows, :] = _lrelu(
            _dot(svu_ref[...], ys_ref[:, :D]) + b1_ref[...])
        tho_ref[rows, :] = _lrelu(
            _dot(tvu_ref[...], yt_ref[:, :D]) + b2_ref[...])

    @pl.when(stage == 1)
    def _():
        @pl.when(i == 0)
        def _():
            ys_ref[:, :D] = _dot(sho_ref[...], w3_ref[...])
            yt_ref[:, :D] = _dot(tho_ref[...], w4_ref[...])

        s_ho2 = _lrelu(_dot(suv_ref[...], ys_ref[:, :D]) + b3_ref[...])
        t_ho2 = _lrelu(_dot(tuv_ref[...], yt_ref[:, :D]) + b4_ref[...])
        sU = (_dot(s_ho2, wsut_ref[...])
              + _dot(s_ref[rows, :], wsub_ref[...]) + bsu_ref[...])
        tU = (_dot(t_ho2, wtut_ref[...])
              + _dot(t_ref[rows, :], wtub_ref[...]) + btu_ref[...])
        mix_ref[rows, :] = jnp.maximum(sU, 0.0) + jnp.maximum(tU, 0.0)

    @pl.when(stage == 2)
    def _():
        @pl.when(i == 0)
        def _():
            ys_ref[:, :D] = _dot(mix_ref[...], w5_ref[...])
            yt_ref[:, :D] = _dot(mix_ref[...], w6_ref[...])

        bb1_ref[rows, :] = _lrelu(
            _dot(svu_ref[...], ys_ref[:, :D]) + b5_ref[...])
        bb2_ref[rows, :] = _lrelu(
            _dot(tvu_ref[...], yt_ref[:, :D]) + b6_ref[...])

    @pl.when(stage == 3)
    def _():
        @pl.when(i == 0)
        def _():
            ys_ref[...] = _dot(bb1_ref[...], w3ms_ref[...])
            yt_ref[...] = _dot(bb2_ref[...], w4ms_ref[...])

        sml = _lrelu(_dot(suv_ref[...], ys_ref[...]) + b3ms_ref[...])
        tml = _lrelu(_dot(tuv_ref[...], yt_ref[...]) + b4ms_ref[...])
        mixv = mix_ref[rows, :]
        mean_ref[...] = (_dot(sml[:, :D], wsm_ref[...])
                         + _dot(tml[:, :D], wtm_ref[...])
                         + _dot(mixv, wmixm_ref[...]) + bm_ref[...])
        logstd_ref[...] = (_dot(sml[:, D:], wsl_ref[...])
                          + _dot(tml[:, D:], wtl_ref[...])
                          + _dot(mixv, wmixl_ref[...]) + bl_ref[...])


def _vu_map(g):
    stage, i = g // GB, g % GB
    return (jnp.where((stage == 0) | (stage == 2), i, GB - 1), 0)


def _uv_map(g):
    stage, i = g // GB, g % GB
    return (jnp.where((stage == 1) | (stage == 3), i,
                      jnp.where(stage == 0, 0, GB - 1)), 0)


def _out_map(g):
    stage, i = g // GB, g % GB
    return (jnp.where(stage == 3, i, 0), 0)


def _const(shape):
    return pl.BlockSpec(shape, lambda g: (0, 0))


def kernel(source_ufea, target_ufea, source_UV_adj, source_VU_adj,
           target_UV_adj, target_VU_adj, source_rate,
           L0_gc1_W, L0_gc1_b, L0_gc2_W, L0_gc2_b, L0_gc3_W, L0_gc3_b,
           L0_gc4_W, L0_gc4_b, L0_su_W, L0_su_b, L0_tu_W, L0_tu_b,
           L1_gc1_W, L1_gc1_b, L1_gc2_W, L1_gc2_b, L1_gc3m_W, L1_gc3m_b,
           L1_gc3s_W, L1_gc3s_b, L1_gc4m_W, L1_gc4m_b, L1_gc4s_W,
           L1_gc4s_b, L1_sum_W, L1_sum_b, L1_sus_W, L1_sus_b, L1_tum_W,
           L1_tum_b, L1_tus_W, L1_tus_b):
    r = source_rate[0]
    rc = 1.0 - r
    row = lambda b: b.reshape(1, -1)

    w3ms = jnp.concatenate([L1_gc3m_W, L1_gc3s_W], axis=1)
    b3ms = jnp.concatenate([L1_gc3m_b, L1_gc3s_b])
    w4ms = jnp.concatenate([L1_gc4m_W, L1_gc4s_W], axis=1)
    b4ms = jnp.concatenate([L1_gc4m_b, L1_gc4s_b])

    dd = _const((D, D))
    db = _const((1, D))
    adj_vu = pl.BlockSpec((BM, N), _vu_map)
    adj_uv = pl.BlockSpec((BM, N), _uv_map)

    mean, logstd = pl.pallas_call(
        _body,
        grid=(4 * GB,),
        in_specs=[
            _const((N, D)), _const((N, D)),
            adj_vu, adj_vu,
            adj_uv, adj_uv,
            dd, db, dd, db,
            dd, db, dd, db,
            dd, dd, db, dd, dd, db,
            dd, db, dd, db,
            _const((D, 2 * D)), _const((1, 2 * D)),
            _const((D, 2 * D)), _const((1, 2 * D)),
            dd, dd, dd, db,
            dd, dd, dd, db,
        ],
        out_specs=[pl.BlockSpec((BM, D), _out_map)] * 2,
        out_shape=[jax.ShapeDtypeStruct((N, D), F32)] * 2,
        scratch_shapes=[
            pltpu.VMEM((N, D), F32),
            pltpu.VMEM((N, D), F32),
            pltpu.VMEM((N, D), F32),
            pltpu.VMEM((N, D), F32),
            pltpu.VMEM((N, D), F32),
            pltpu.VMEM((N, 2 * D), F32),
            pltpu.VMEM((N, 2 * D), F32),
        ],
        compiler_params=pltpu.CompilerParams(
            dimension_semantics=("arbitrary",),
            vmem_limit_bytes=64 * 1024 * 1024,
        ),
    )(source_ufea, target_ufea, source_VU_adj, target_VU_adj,
      source_UV_adj, target_UV_adj,
      L0_gc1_W, row(L0_gc1_b), L0_gc2_W, row(L0_gc2_b),
      L0_gc3_W, row(L0_gc3_b), L0_gc4_W, row(L0_gc4_b),
      r * L0_su_W[:D], r * L0_su_W[D:], row(r * L0_su_b),
      rc * L0_tu_W[:D], rc * L0_tu_W[D:], row(rc * L0_tu_b),
      L1_gc1_W, row(L1_gc1_b), L1_gc2_W, row(L1_gc2_b),
      w3ms, row(b3ms), w4ms, row(b4ms),
      r * L1_sum_W[:D], rc * L1_tum_W[:D],
      r * L1_sum_W[D:] + rc * L1_tum_W[D:],
      row(r * L1_sum_b + rc * L1_tum_b),
      r * L1_sus_W[:D], rc * L1_tus_W[:D],
      r * L1_sus_W[D:] + rc * L1_tus_W[D:],
      row(r * L1_sus_b + rc * L1_tus_b))
    return (mean, logstd)
